# baseline (speedup 1.0000x reference)
import functools

import jax
import jax.numpy as jnp
import numpy as np
from jax.experimental import pallas as pl

H, D = 8, 16
HID = 128
OUT = 64
NT = ['job', 'user']
ETS = [('job', 'similar_to', 'job'), ('user', 'applied', 'job'), ('job', 'rev_applied', 'user')]


def _pred_mlp_body(emb_ref, w1_ref, b1_ref, w2_ref, b2_ref, out_ref):
    h = jnp.maximum(emb_ref[...] @ w1_ref[...] + b1_ref[...], 0.0)
    out_ref[...] = h @ w2_ref[...] + b2_ref[...]


def _pred_mlp(emb, w1, b1, w2, b2):
    n = emb.shape[0]
    blk = 2000
    grid = (n // blk,)
    return pl.pallas_call(
        _pred_mlp_body,
        grid=grid,
        in_specs=[
            pl.BlockSpec((blk, 2 * OUT), lambda i: (i, 0)),
            pl.BlockSpec((2 * OUT, HID), lambda i: (0, 0)),
            pl.BlockSpec((1, HID), lambda i: (0, 0)),
            pl.BlockSpec((HID, 1), lambda i: (0, 0)),
            pl.BlockSpec((1, 1), lambda i: (0, 0)),
        ],
        out_specs=pl.BlockSpec((blk, 1), lambda i: (i, 0)),
        out_shape=jax.ShapeDtypeStruct((n, 1), jnp.float32),
    )(emb, w1, b1[None, :], w2, b2[None, :])


def _hgt_conv(x, ei, p, l):
    k = {nt: (x[nt] @ p['l%d_%s_Wk' % (l, nt)] + p['l%d_%s_bk' % (l, nt)]).reshape(-1, H, D) for nt in NT}
    q = {nt: (x[nt] @ p['l%d_%s_Wq' % (l, nt)] + p['l%d_%s_bq' % (l, nt)]).reshape(-1, H, D) for nt in NT}
    v = {nt: (x[nt] @ p['l%d_%s_Wv' % (l, nt)] + p['l%d_%s_bv' % (l, nt)]).reshape(-1, H, D) for nt in NT}
    agg = {nt: jnp.zeros((x[nt].shape[0], H, D), jnp.float32) for nt in NT}
    for (st, rel, dt) in ETS:
        e = ei[rel]
        si, di = e[0], e[1]
        k_rel = jnp.einsum('nhd,hde->nhe', k[st], p['l%d_%s_a' % (l, rel)])
        v_rel = jnp.einsum('nhd,hde->nhe', v[st], p['l%d_%s_m' % (l, rel)])
        nd = x[dt].shape[0]
        alpha = (q[dt][di] * k_rel[si]).sum(-1) * p['l%d_%s_p' % (l, rel)] / np.sqrt(D)
        amax = jax.ops.segment_max(alpha, di, num_segments=nd)
        amax = jnp.where(jnp.isfinite(amax), amax, 0.0)
        ex = jnp.exp(alpha - amax[di])
        den = jax.ops.segment_sum(ex, di, num_segments=nd)
        wgt = ex / (den[di] + 1e-16)
        msg = v_rel[si] * wgt[..., None]
        agg[dt] = agg[dt] + jax.ops.segment_sum(msg, di, num_segments=nd)
    new = {}
    for nt in NT:
        o = agg[nt].reshape(-1, H * D)
        o = jax.nn.gelu(o) @ p['l%d_%s_Wa' % (l, nt)] + p['l%d_%s_ba' % (l, nt)]
        s = jax.nn.sigmoid(p['l%d_%s_skip' % (l, nt)])
        new[nt] = s * o + (1.0 - s) * x[nt]
    return new


def kernel(x_job, x_user, edge_index_sim, edge_index_app, edge_index_rev, edge_label_index, params):
    p = params
    ei = {'similar_to': edge_index_sim, 'applied': edge_index_app, 'rev_applied': edge_index_rev}
    x = {'job': x_job, 'user': x_user}
    x = {nt: jax.nn.relu(x[nt] @ p['in_' + nt + '_W'] + p['in_' + nt + '_b']) for nt in NT}
    for l in range(2):
        x = _hgt_conv(x, ei, p, l)
        x = {nt: jax.nn.relu(xv) for nt, xv in x.items()}
    out_job = x['job'] @ p['out_W'] + p['out_b']
    src = out_job[edge_label_index[0]]
    dst = out_job[edge_label_index[1]]
    emb = jnp.concatenate([src, dst], axis=-1)
    pred = _pred_mlp(emb, p['pred_W1'], p['pred_b1'], p['pred_W2'], p['pred_b2'])[:, 0]
    return pred


# speedup vs baseline: 14.1930x; 14.1930x over previous
import functools

import jax
import jax.numpy as jnp
from jax import lax
from jax.experimental import pallas as pl
from jax.experimental.pallas import tpu as pltpu
from jax.experimental.pallas import tpu_sc as plsc

H, D = 8, 16
HID = 128
OUT = 64
NC, NS = 2, 16
N1J = 50176
N1U = 10240
NJOB, NUSER = 50000, 10000
F32 = jnp.float32
I32 = jnp.int32


def _mm_body(x_ref, w_ref, b_ref, *o_refs, act, fout):
    y = jnp.dot(x_ref[...], w_ref[...], preferred_element_type=F32) + b_ref[...]
    if act:
        y = jnp.maximum(y, 0.0)
    for j, o in enumerate(o_refs):
        o[...] = y[:, j * fout:(j + 1) * fout]


def _mm(x, w, b, act=False, nout=1, blk=1024):
    n, kdim = x.shape
    f = w.shape[1]
    fout = f // nout
    outs = pl.pallas_call(
        functools.partial(_mm_body, act=act, fout=fout),
        grid=(n // blk,),
        in_specs=[
            pl.BlockSpec((blk, kdim), lambda i: (i, 0)),
            pl.BlockSpec((kdim, f), lambda i: (0, 0)),
            pl.BlockSpec((1, f), lambda i: (0, 0)),
        ],
        out_specs=[pl.BlockSpec((blk, fout), lambda i: (i, 0))] * nout,
        out_shape=[jax.ShapeDtypeStruct((n, fout), F32)] * nout,
    )(x, w, b[None])
    return outs if nout > 1 else outs[0]


def _post_body(h_ref, wa_ref, ba_ref, s_ref, *rest, ng, nrel):
    fg = HID // ng
    hg = H // ng
    refs = rest[:-1]
    o_ref = rest[-1]
    wa = wa_ref[...]
    s = s_ref[0, 0]
    blk = h_ref.shape[0]
    o = jnp.zeros((blk, HID), F32)
    for g in range(ng):
        agg = jnp.zeros((blk, fg), F32)
        for r in range(nrel):
            num = refs[2 * r][g]
            den = refs[2 * r + 1][g]
            den_x = jnp.concatenate(
                [jnp.broadcast_to(den[:, h:h + 1], (blk, D)) for h in range(hg)],
                axis=1)
            agg = agg + num / (den_x + 1e-16)
        o = o + jnp.dot(jax.nn.gelu(agg), wa[g * fg:(g + 1) * fg, :],
                        preferred_element_type=F32)
    o = o + ba_ref[...]
    o_ref[...] = jnp.maximum(s * o + (1.0 - s) * h_ref[...], 0.0)


def _post(h, wa, ba, s, aggs, ng, blk=1024):
    n = h.shape[0]
    fg = HID // ng
    hg = H // ng
    nrel = len(aggs) // 2
    agg_specs = []
    for r in range(nrel):
        agg_specs.append(pl.BlockSpec((ng, blk, fg), lambda i: (0, i, 0)))
        agg_specs.append(pl.BlockSpec((ng, blk, hg), lambda i: (0, i, 0)))
    return pl.pallas_call(
        functools.partial(_post_body, ng=ng, nrel=nrel),
        grid=(n // blk,),
        in_specs=[
            pl.BlockSpec((blk, HID), lambda i: (i, 0)),
            pl.BlockSpec((HID, HID), lambda i: (0, 0)),
            pl.BlockSpec((1, HID), lambda i: (0, 0)),
            pl.BlockSpec((1, 1), lambda i: (0, 0)),
        ] + agg_specs,
        out_specs=pl.BlockSpec((blk, HID), lambda i: (i, 0)),
        out_shape=jax.ShapeDtypeStruct((n, HID), F32),
    )(h, wa, ba[None], s, *aggs)


def _pred_body(a_ref, b_ref, bt_ref, w2_ref, b2_ref, o_ref):
    z = jnp.maximum(a_ref[...] + b_ref[...] + bt_ref[...], 0.0)
    o_ref[...] = jnp.dot(z, w2_ref[...], preferred_element_type=F32) + b2_ref[...]


def _pred(a, b, bt, w2, b2, blk=2048):
    n = a.shape[0]
    return pl.pallas_call(
        _pred_body,
        grid=(n // blk,),
        in_specs=[
            pl.BlockSpec((blk, HID), lambda i: (i, 0)),
            pl.BlockSpec((blk, HID), lambda i: (i, 0)),
            pl.BlockSpec((1, HID), lambda i: (0, 0)),
            pl.BlockSpec((HID, 1), lambda i: (0, 0)),
            pl.BlockSpec((1, 1), lambda i: (0, 0)),
        ],
        out_specs=pl.BlockSpec((blk, 1), lambda i: (i, 0)),
        out_shape=jax.ShapeDtypeStruct((n, 1), F32),
    )(a, b, bt[None], w2, b2[None])


@functools.lru_cache(maxsize=None)
def _make_edge_call(ep, nsrc1, ndst1, ng, c=64):
    fg = HID // ng
    hg = H // ng
    slab = ndst1 // NS
    dtab = ndst1 * hg // 16
    dslab = dtab // NS
    nchunks = ep // (NS * c)
    npc = ng // NC
    mesh = plsc.VectorSubcoreMesh(core_axis_name="c", subcore_axis_name="s",
                                  num_cores=NC, num_subcores=NS)

    @functools.partial(
        pl.kernel,
        out_type=(jax.ShapeDtypeStruct((ng, ndst1, fg), F32),
                  jax.ShapeDtypeStruct((ng, dtab, 16), F32)),
        mesh=mesh,
        compiler_params=pltpu.CompilerParams(needs_layout_passes=False, use_tc_tiling_on_sc=False),
        scratch_types=[
            pltpu.VMEM_SHARED((ndst1, fg), F32),
            pltpu.VMEM_SHARED((dtab, 16), F32),
            pltpu.VMEM((c,), I32),
            pltpu.VMEM((c,), I32),
            pltpu.VMEM((c,), I32),
            pltpu.VMEM((c,), I32),
            pltpu.VMEM((c,), I32),
            pltpu.VMEM((c, fg), F32),
            pltpu.VMEM((c, fg), F32),
            pltpu.VMEM((c, fg), F32),
            pltpu.VMEM((c, fg), F32),
            pltpu.VMEM((c, 16), F32),
            pltpu.VMEM((64, fg), F32),
            pltpu.SemaphoreType.DMA,
            pltpu.SemaphoreType.DMA,
            pltpu.SemaphoreType.DMA,
        ],
    )
    def edge_kernel(qt, kt, vt, si, di, zden, num_out, den_out,
                    num_sh, den_sh, siv, div_, idxq, idxk, idxd,
                    qb, kb, vb, msgb, denw, zbuf, sem1, sem2, sem3):
        cid = lax.axis_index("c")
        sid = lax.axis_index("s")
        iota = lax.iota(I32, 16)
        zv = jnp.zeros((16,), F32)
        for r in range(64):
            for c0 in range(fg // 16):
                zbuf[r, pl.ds(c0 * 16, 16)] = zv
        r0 = sid * slab
        for p_i in range(npc):
            g = cid * npc + p_i
            for kk in range(slab // 64):
                pltpu.sync_copy(zbuf, num_sh.at[pl.ds(r0 + kk * 64, 64)])
            pltpu.sync_copy(zden, den_sh.at[pl.ds(sid * dslab, dslab)])
            plsc.subcore_barrier()

            def chunk_body(i, _, g=g):
                base = (i * NS + sid) * c
                pltpu.sync_copy(si.at[pl.ds(base, c)], siv)
                pltpu.sync_copy(di.at[pl.ds(base, c)], div_)
                for j in range(c // 16):
                    s16 = siv[pl.ds(j * 16, 16)]
                    d16 = div_[pl.ds(j * 16, 16)]
                    idxk[pl.ds(j * 16, 16)] = s16 * ng + g
                    idxq[pl.ds(j * 16, 16)] = d16 * ng + g
                    idxd[pl.ds(j * 16, 16)] = (d16 * hg) >> 4
                cp1 = pltpu.async_copy(qt.at[idxq], qb, sem1)
                cp2 = pltpu.async_copy(kt.at[idxk], kb, sem2)
                cp3 = pltpu.async_copy(vt.at[idxk], vb, sem3)
                cp1.wait()
                cp2.wait()
                cp3.wait()
                for r in range(c):
                    denw[r, pl.ds(0, 16)] = zv
                for t in range(c // 16):
                    rows = t * 16 + iota
                    d16 = div_[pl.ds(t * 16, 16)]
                    dbase = (d16 * hg) & 15
                    for h in range(hg):
                        acc = jnp.zeros((16,), F32)
                        for d0 in range(D):
                            col = jnp.full((16,), h * D + d0, I32)
                            acc = acc + (plsc.load_gather(qb, [rows, col])
                                         * plsc.load_gather(kb, [rows, col]))
                        ex = jnp.exp(acc)
                        plsc.store_scatter(denw, [rows, dbase + h], ex)
                        for d0 in range(D):
                            col = jnp.full((16,), h * D + d0, I32)
                            mv = plsc.load_gather(vb, [rows, col]) * ex
                            plsc.store_scatter(msgb, [rows, col], mv)
                pltpu.sync_copy(msgb, num_sh.at[div_], add=True)
                pltpu.sync_copy(denw, den_sh.at[idxd], add=True)
                return 0

            lax.fori_loop(0, nchunks, chunk_body, 0)
            plsc.subcore_barrier()
            pltpu.sync_copy(num_sh.at[pl.ds(r0, slab)],
                            num_out.at[g, pl.ds(r0, slab)])
            pltpu.sync_copy(den_sh.at[pl.ds(sid * dslab, dslab)],
                            den_out.at[g, pl.ds(sid * dslab, dslab)])
            plsc.subcore_barrier()

    return edge_kernel


def _edge_conv(q, k, v, si_p, di_p, nsrc1, ndst1, ng):
    ep = si_p.shape[0]
    fg = HID // ng
    hg = H // ng
    fn = _make_edge_call(ep, nsrc1, ndst1, ng)
    zden = jnp.zeros((ndst1 * hg // 16 // NS, 16), F32)
    qv = q.reshape(ndst1 * ng, fg)
    kv = k.reshape(nsrc1 * ng, fg)
    vv = v.reshape(nsrc1 * ng, fg)
    num, den = fn(qv, kv, vv, si_p, di_p, zden)
    return num, den.reshape(ng, ndst1, hg)


@functools.lru_cache(maxsize=None)
def _make_pair_gather(ep, n1, c=128):
    nchunks = ep // (NS * NC * c)
    mesh = plsc.VectorSubcoreMesh(core_axis_name="c", subcore_axis_name="s",
                                  num_cores=NC, num_subcores=NS)

    @functools.partial(
        pl.kernel,
        out_type=(jax.ShapeDtypeStruct((ep, HID), F32),
                  jax.ShapeDtypeStruct((ep, HID), F32)),
        mesh=mesh,
        compiler_params=pltpu.CompilerParams(needs_layout_passes=False, use_tc_tiling_on_sc=False),
        scratch_types=[
            pltpu.VMEM((c,), I32),
            pltpu.VMEM((c,), I32),
            pltpu.VMEM((c, HID), F32),
            pltpu.VMEM((c, HID), F32),
            pltpu.SemaphoreType.DMA,
            pltpu.SemaphoreType.DMA,
        ],
    )
    def pair_kernel(g1t, g2t, src, dst, a_out, b_out,
                    siv, div_, ab, bb, sem1, sem2):
        wid = lax.axis_index("s") * NC + lax.axis_index("c")

        def chunk_body(i, _):
            base = (i * NS * NC + wid) * c
            pltpu.sync_copy(src.at[pl.ds(base, c)], siv)
            pltpu.sync_copy(dst.at[pl.ds(base, c)], div_)
            cp1 = pltpu.async_copy(g1t.at[siv], ab, sem1)
            cp2 = pltpu.async_copy(g2t.at[div_], bb, sem2)
            cp1.wait()
            cp2.wait()
            pltpu.sync_copy(ab, a_out.at[pl.ds(base, c)])
            pltpu.sync_copy(bb, b_out.at[pl.ds(base, c)])
            return 0

        lax.fori_loop(0, nchunks, chunk_body, 0)

    return pair_kernel


def _bdiag(a):
    out = jnp.zeros((HID, HID), F32)
    for h in range(H):
        out = out.at[h * D:(h + 1) * D, h * D:(h + 1) * D].set(a[h])
    return out


def _pad_edges(e, ep, spad, dpad):
    n = e.shape[1]
    si = jnp.concatenate([e[0], jnp.full((ep - n,), spad, I32)])
    di = jnp.concatenate([e[1], jnp.full((ep - n,), dpad, I32)])
    return si, di


def kernel(x_job, x_user, edge_index_sim, edge_index_app, edge_index_rev,
           edge_label_index, params):
    p = params
    sqd = jnp.sqrt(jnp.float32(D))

    def kv_w(l, st, rel):
        a_s = p['l%d_%s_a' % (l, rel)] * (p['l%d_%s_p' % (l, rel)] / sqd)[:, None, None]
        abd = _bdiag(a_s)
        mbd = _bdiag(p['l%d_%s_m' % (l, rel)])
        wk = p['l%d_%s_Wk' % (l, st)] @ abd
        bk = p['l%d_%s_bk' % (l, st)] @ abd
        wv = p['l%d_%s_Wv' % (l, st)] @ mbd
        bv = p['l%d_%s_bv' % (l, st)] @ mbd
        return wk, bk, wv, bv

    xj = jnp.pad(x_job, ((0, N1J - NJOB), (0, 0)))
    xu = jnp.pad(x_user, ((0, N1U - NUSER), (0, 0)))

    si_sim, di_sim = _pad_edges(edge_index_sim, 401408, NJOB, NJOB)
    si_app, di_app = _pad_edges(edge_index_app, 106496, NUSER, NJOB)
    si_rev, di_rev = _pad_edges(edge_index_rev, 106496, NJOB, NUSER)

    h_job = _mm(xj, p['in_job_W'], p['in_job_b'], act=True)
    h_user = _mm(xu, p['in_user_W'], p['in_user_b'], act=True)

    for l in range(2):
        wk_s, bk_s, wv_s, bv_s = kv_w(l, 'job', 'similar_to')
        wk_a, bk_a, wv_a, bv_a = kv_w(l, 'user', 'applied')
        wj = [p['l%d_job_Wq' % l], wk_s, wv_s]
        bj = [p['l%d_job_bq' % l], bk_s, bv_s]
        wu = [wk_a, wv_a]
        bu = [bk_a, bv_a]
        if l == 0:
            wk_r, bk_r, wv_r, bv_r = kv_w(l, 'job', 'rev_applied')
            wj += [wk_r, wv_r]
            bj += [bk_r, bv_r]
            wu = [p['l%d_user_Wq' % l]] + wu
            bu = [p['l%d_user_bq' % l]] + bu
        outs_j = _mm(h_job, jnp.concatenate(wj, axis=1),
                     jnp.concatenate(bj), nout=len(wj))
        outs_u = _mm(h_user, jnp.concatenate(wu, axis=1),
                     jnp.concatenate(bu), nout=len(wu))
        if l == 0:
            q_j, k_s, v_s, k_r, v_r = outs_j
            q_u, k_a, v_a = outs_u
        else:
            q_j, k_s, v_s = outs_j
            k_a, v_a = outs_u

        num_s, den_s = _edge_conv(q_j, k_s, v_s, si_sim, di_sim, N1J, N1J, 4)
        num_a, den_a = _edge_conv(q_j, k_a, v_a, si_app, di_app, N1U, N1J, 4)
        s_j = jax.nn.sigmoid(p['l%d_job_skip' % l]).reshape(1, 1)
        h_job = _post(h_job, p['l%d_job_Wa' % l], p['l%d_job_ba' % l], s_j,
                      [num_s, den_s, num_a, den_a], ng=4)
        if l == 0:
            num_r, den_r = _edge_conv(q_u, k_r, v_r, si_rev, di_rev, N1J, N1U, 2)
            s_u = jax.nn.sigmoid(p['l%d_user_skip' % l]).reshape(1, 1)
            h_user = _post(h_user, p['l%d_user_Wa' % l], p['l%d_user_ba' % l],
                           s_u, [num_r, den_r], ng=2)

    w1a, w1b = p['pred_W1'][:OUT], p['pred_W1'][OUT:]
    wg1 = p['out_W'] @ w1a
    wg2 = p['out_W'] @ w1b
    btot = p['out_b'] @ w1a + p['out_b'] @ w1b + p['pred_b1']
    g1, g2 = _mm(h_job, jnp.concatenate([wg1, wg2], axis=1),
                 jnp.zeros((2 * HID,), F32), nout=2)

    epl = 204800
    nlab = edge_label_index.shape[1]
    src = jnp.concatenate([edge_label_index[0], jnp.full((epl - nlab,), NJOB, I32)])
    dst = jnp.concatenate([edge_label_index[1], jnp.full((epl - nlab,), NJOB, I32)])
    ga, gb = _make_pair_gather(epl, N1J)(g1, g2, src, dst)
    pred = _pred(ga, gb, btot, p['pred_W2'], p['pred_b2'])
    return pred[:nlab, 0]


# speedup vs baseline: 15.5129x; 1.0930x over previous
import functools

import jax
import jax.numpy as jnp
from jax import lax
from jax.experimental import pallas as pl
from jax.experimental.pallas import tpu as pltpu
from jax.experimental.pallas import tpu_sc as plsc

H, D = 8, 16
HID = 128
OUT = 64
NC, NS = 2, 16
N1J = 50176
N1U = 10240
NJOB, NUSER = 50000, 10000
F32 = jnp.float32
I32 = jnp.int32


def _mm_body(x_ref, w_ref, b_ref, *o_refs, act, fout):
    y = jnp.dot(x_ref[...], w_ref[...], preferred_element_type=F32) + b_ref[...]
    if act:
        y = jnp.maximum(y, 0.0)
    for j, o in enumerate(o_refs):
        o[...] = y[:, j * fout:(j + 1) * fout]


def _mm(x, w, b, act=False, nout=1, blk=1024):
    n, kdim = x.shape
    f = w.shape[1]
    fout = f // nout
    outs = pl.pallas_call(
        functools.partial(_mm_body, act=act, fout=fout),
        grid=(n // blk,),
        in_specs=[
            pl.BlockSpec((blk, kdim), lambda i: (i, 0)),
            pl.BlockSpec((kdim, f), lambda i: (0, 0)),
            pl.BlockSpec((1, f), lambda i: (0, 0)),
        ],
        out_specs=[pl.BlockSpec((blk, fout), lambda i: (i, 0))] * nout,
        out_shape=[jax.ShapeDtypeStruct((n, fout), F32)] * nout,
    )(x, w, b[None])
    return outs if nout > 1 else outs[0]


def _post_body(h_ref, wa_ref, ba_ref, s_ref, *rest, ng, nrel):
    fg = HID // ng
    hg = H // ng
    refs = rest[:-1]
    o_ref = rest[-1]
    wa = wa_ref[...]
    s = s_ref[0, 0]
    blk = h_ref.shape[0]
    o = jnp.zeros((blk, HID), F32)
    for g in range(ng):
        agg = jnp.zeros((blk, fg), F32)
        for r in range(nrel):
            num = refs[2 * r][g]
            den = refs[2 * r + 1][g]
            den_x = jnp.concatenate(
                [jnp.broadcast_to(den[:, h:h + 1], (blk, D)) for h in range(hg)],
                axis=1)
            agg = agg + num / (den_x + 1e-16)
        o = o + jnp.dot(jax.nn.gelu(agg), wa[g * fg:(g + 1) * fg, :],
                        preferred_element_type=F32)
    o = o + ba_ref[...]
    o_ref[...] = jnp.maximum(s * o + (1.0 - s) * h_ref[...], 0.0)


def _post(h, wa, ba, s, aggs, ng, blk=1024):
    n = h.shape[0]
    fg = HID // ng
    hg = H // ng
    nrel = len(aggs) // 2
    agg_specs = []
    for r in range(nrel):
        agg_specs.append(pl.BlockSpec((ng, blk, fg), lambda i: (0, i, 0)))
        agg_specs.append(pl.BlockSpec((ng, blk, hg), lambda i: (0, i, 0)))
    return pl.pallas_call(
        functools.partial(_post_body, ng=ng, nrel=nrel),
        grid=(n // blk,),
        in_specs=[
            pl.BlockSpec((blk, HID), lambda i: (i, 0)),
            pl.BlockSpec((HID, HID), lambda i: (0, 0)),
            pl.BlockSpec((1, HID), lambda i: (0, 0)),
            pl.BlockSpec((1, 1), lambda i: (0, 0)),
        ] + agg_specs,
        out_specs=pl.BlockSpec((blk, HID), lambda i: (i, 0)),
        out_shape=jax.ShapeDtypeStruct((n, HID), F32),
    )(h, wa, ba[None], s, *aggs)


def _pred_body(a_ref, b_ref, bt_ref, w2_ref, b2_ref, o_ref):
    z = jnp.maximum(a_ref[...] + b_ref[...] + bt_ref[...], 0.0)
    o_ref[...] = jnp.dot(z, w2_ref[...], preferred_element_type=F32) + b2_ref[...]


def _pred(a, b, bt, w2, b2, blk=2048):
    n = a.shape[0]
    return pl.pallas_call(
        _pred_body,
        grid=(n // blk,),
        in_specs=[
            pl.BlockSpec((blk, HID), lambda i: (i, 0)),
            pl.BlockSpec((blk, HID), lambda i: (i, 0)),
            pl.BlockSpec((1, HID), lambda i: (0, 0)),
            pl.BlockSpec((HID, 1), lambda i: (0, 0)),
            pl.BlockSpec((1, 1), lambda i: (0, 0)),
        ],
        out_specs=pl.BlockSpec((blk, 1), lambda i: (i, 0)),
        out_shape=jax.ShapeDtypeStruct((n, 1), F32),
    )(a, b, bt[None], w2, b2[None])


@functools.lru_cache(maxsize=None)
def _make_edge_call(ep, nsrc1, ndst1, ng, c=128):
    fg = HID // ng
    hg = H // ng
    slab = ndst1 // NS
    dtab = ndst1 * hg // 16
    dslab = dtab // NS
    nchunks = ep // (NS * c)
    npc = ng // NC
    mesh = plsc.VectorSubcoreMesh(core_axis_name="c", subcore_axis_name="s",
                                  num_cores=NC, num_subcores=NS)

    @functools.partial(
        pl.kernel,
        out_type=(jax.ShapeDtypeStruct((ng, ndst1, fg), F32),
                  jax.ShapeDtypeStruct((ng, dtab, 16), F32)),
        mesh=mesh,
        compiler_params=pltpu.CompilerParams(needs_layout_passes=False, use_tc_tiling_on_sc=False),
        scratch_types=[
            pltpu.VMEM_SHARED((ndst1, fg), F32),
            pltpu.VMEM_SHARED((dtab, 16), F32),
            pltpu.VMEM((c,), I32),
            pltpu.VMEM((c,), I32),
            pltpu.VMEM((c,), I32),
            pltpu.VMEM((c,), I32),
            pltpu.VMEM((c,), I32),
            pltpu.VMEM((c, fg), F32),
            pltpu.VMEM((c, fg), F32),
            pltpu.VMEM((c, fg), F32),
            pltpu.VMEM((c, fg), F32),
            pltpu.VMEM((c, 16), F32),
            pltpu.VMEM((64, fg), F32),
            pltpu.SemaphoreType.DMA,
            pltpu.SemaphoreType.DMA,
            pltpu.SemaphoreType.DMA,
        ],
    )
    def edge_kernel(qt, kt, vt, si, di, zden, num_out, den_out,
                    num_sh, den_sh, siv, div_, idxq, idxk, idxd,
                    qb, kb, vb, msgb, denw, zbuf, sem1, sem2, sem3):
        cid = lax.axis_index("c")
        sid = lax.axis_index("s")
        iota = lax.iota(I32, 16)
        zv = jnp.zeros((16,), F32)
        for r in range(64):
            for c0 in range(fg // 16):
                zbuf[r, pl.ds(c0 * 16, 16)] = zv
        r0 = sid * slab
        for p_i in range(npc):
            g = cid * npc + p_i
            for kk in range(slab // 64):
                pltpu.sync_copy(zbuf, num_sh.at[pl.ds(r0 + kk * 64, 64)])
            pltpu.sync_copy(zden, den_sh.at[pl.ds(sid * dslab, dslab)])
            plsc.subcore_barrier()

            def chunk_body(i, _, g=g):
                base = (i * NS + sid) * c
                pltpu.sync_copy(si.at[pl.ds(base, c)], siv)
                pltpu.sync_copy(di.at[pl.ds(base, c)], div_)
                for j in range(c // 16):
                    s16 = siv[pl.ds(j * 16, 16)]
                    d16 = div_[pl.ds(j * 16, 16)]
                    idxk[pl.ds(j * 16, 16)] = s16 * ng + g
                    idxq[pl.ds(j * 16, 16)] = d16 * ng + g
                    idxd[pl.ds(j * 16, 16)] = (d16 * hg) >> 4
                cp1 = pltpu.async_copy(qt.at[idxq], qb, sem1)
                cp2 = pltpu.async_copy(kt.at[idxk], kb, sem2)
                cp3 = pltpu.async_copy(vt.at[idxk], vb, sem3)
                cp1.wait()
                cp2.wait()
                cp3.wait()

                def grp_body(t, _):
                    rows = t * 16 + iota
                    d16 = plsc.load_gather(div_, [rows])
                    dbase = (d16 * hg) & 15
                    for w in range(16):
                        plsc.store_scatter(denw, [rows, jnp.full((16,), w, I32)], zv)
                    for h in range(hg):
                        acc = jnp.zeros((16,), F32)
                        for d0 in range(D):
                            col = jnp.full((16,), h * D + d0, I32)
                            acc = acc + (plsc.load_gather(qb, [rows, col])
                                         * plsc.load_gather(kb, [rows, col]))
                        ex = jnp.exp(acc)
                        plsc.store_scatter(denw, [rows, dbase + h], ex)
                        for d0 in range(D):
                            col = jnp.full((16,), h * D + d0, I32)
                            mv = plsc.load_gather(vb, [rows, col]) * ex
                            plsc.store_scatter(msgb, [rows, col], mv)
                    return 0

                lax.fori_loop(0, c // 16, grp_body, 0)
                pltpu.sync_copy(msgb, num_sh.at[div_], add=True)
                pltpu.sync_copy(denw, den_sh.at[idxd], add=True)
                return 0

            lax.fori_loop(0, nchunks, chunk_body, 0)
            plsc.subcore_barrier()
            pltpu.sync_copy(num_sh.at[pl.ds(r0, slab)],
                            num_out.at[g, pl.ds(r0, slab)])
            pltpu.sync_copy(den_sh.at[pl.ds(sid * dslab, dslab)],
                            den_out.at[g, pl.ds(sid * dslab, dslab)])
            plsc.subcore_barrier()

    return edge_kernel


def _edge_conv(q, k, v, si_p, di_p, nsrc1, ndst1, ng):
    ep = si_p.shape[0]
    fg = HID // ng
    hg = H // ng
    fn = _make_edge_call(ep, nsrc1, ndst1, ng)
    zden = jnp.zeros((ndst1 * hg // 16 // NS, 16), F32)
    qv = q.reshape(ndst1 * ng, fg)
    kv = k.reshape(nsrc1 * ng, fg)
    vv = v.reshape(nsrc1 * ng, fg)
    num, den = fn(qv, kv, vv, si_p, di_p, zden)
    return num, den.reshape(ng, ndst1, hg)


@functools.lru_cache(maxsize=None)
def _make_pair_gather(ep, n1, c=128):
    nchunks = ep // (NS * NC * c)
    mesh = plsc.VectorSubcoreMesh(core_axis_name="c", subcore_axis_name="s",
                                  num_cores=NC, num_subcores=NS)

    @functools.partial(
        pl.kernel,
        out_type=(jax.ShapeDtypeStruct((ep, HID), F32),
                  jax.ShapeDtypeStruct((ep, HID), F32)),
        mesh=mesh,
        compiler_params=pltpu.CompilerParams(needs_layout_passes=False, use_tc_tiling_on_sc=False),
        scratch_types=[
            pltpu.VMEM((c,), I32),
            pltpu.VMEM((c,), I32),
            pltpu.VMEM((c, HID), F32),
            pltpu.VMEM((c, HID), F32),
            pltpu.SemaphoreType.DMA,
            pltpu.SemaphoreType.DMA,
        ],
    )
    def pair_kernel(g1t, g2t, src, dst, a_out, b_out,
                    siv, div_, ab, bb, sem1, sem2):
        wid = lax.axis_index("s") * NC + lax.axis_index("c")

        def chunk_body(i, _):
            base = (i * NS * NC + wid) * c
            pltpu.sync_copy(src.at[pl.ds(base, c)], siv)
            pltpu.sync_copy(dst.at[pl.ds(base, c)], div_)
            cp1 = pltpu.async_copy(g1t.at[siv], ab, sem1)
            cp2 = pltpu.async_copy(g2t.at[div_], bb, sem2)
            cp1.wait()
            cp2.wait()
            pltpu.sync_copy(ab, a_out.at[pl.ds(base, c)])
            pltpu.sync_copy(bb, b_out.at[pl.ds(base, c)])
            return 0

        lax.fori_loop(0, nchunks, chunk_body, 0)

    return pair_kernel


def _bdiag(a):
    out = jnp.zeros((HID, HID), F32)
    for h in range(H):
        out = out.at[h * D:(h + 1) * D, h * D:(h + 1) * D].set(a[h])
    return out


def _pad_edges(e, ep, spad, dpad):
    n = e.shape[1]
    si = jnp.concatenate([e[0], jnp.full((ep - n,), spad, I32)])
    di = jnp.concatenate([e[1], jnp.full((ep - n,), dpad, I32)])
    return si, di


def kernel(x_job, x_user, edge_index_sim, edge_index_app, edge_index_rev,
           edge_label_index, params):
    p = params
    sqd = jnp.sqrt(jnp.float32(D))

    def kv_w(l, st, rel):
        a_s = p['l%d_%s_a' % (l, rel)] * (p['l%d_%s_p' % (l, rel)] / sqd)[:, None, None]
        abd = _bdiag(a_s)
        mbd = _bdiag(p['l%d_%s_m' % (l, rel)])
        wk = p['l%d_%s_Wk' % (l, st)] @ abd
        bk = p['l%d_%s_bk' % (l, st)] @ abd
        wv = p['l%d_%s_Wv' % (l, st)] @ mbd
        bv = p['l%d_%s_bv' % (l, st)] @ mbd
        return wk, bk, wv, bv

    xj = jnp.pad(x_job, ((0, N1J - NJOB), (0, 0)))
    xu = jnp.pad(x_user, ((0, N1U - NUSER), (0, 0)))

    si_sim, di_sim = _pad_edges(edge_index_sim, 401408, NJOB, NJOB)
    si_app, di_app = _pad_edges(edge_index_app, 106496, NUSER, NJOB)
    si_rev, di_rev = _pad_edges(edge_index_rev, 106496, NJOB, NUSER)

    h_job = _mm(xj, p['in_job_W'], p['in_job_b'], act=True)
    h_user = _mm(xu, p['in_user_W'], p['in_user_b'], act=True)

    for l in range(2):
        wk_s, bk_s, wv_s, bv_s = kv_w(l, 'job', 'similar_to')
        wk_a, bk_a, wv_a, bv_a = kv_w(l, 'user', 'applied')
        wj = [p['l%d_job_Wq' % l], wk_s, wv_s]
        bj = [p['l%d_job_bq' % l], bk_s, bv_s]
        wu = [wk_a, wv_a]
        bu = [bk_a, bv_a]
        if l == 0:
            wk_r, bk_r, wv_r, bv_r = kv_w(l, 'job', 'rev_applied')
            wj += [wk_r, wv_r]
            bj += [bk_r, bv_r]
            wu = [p['l%d_user_Wq' % l]] + wu
            bu = [p['l%d_user_bq' % l]] + bu
        outs_j = _mm(h_job, jnp.concatenate(wj, axis=1),
                     jnp.concatenate(bj), nout=len(wj))
        outs_u = _mm(h_user, jnp.concatenate(wu, axis=1),
                     jnp.concatenate(bu), nout=len(wu))
        if l == 0:
            q_j, k_s, v_s, k_r, v_r = outs_j
            q_u, k_a, v_a = outs_u
        else:
            q_j, k_s, v_s = outs_j
            k_a, v_a = outs_u

        num_s, den_s = _edge_conv(q_j, k_s, v_s, si_sim, di_sim, N1J, N1J, 4)
        num_a, den_a = _edge_conv(q_j, k_a, v_a, si_app, di_app, N1U, N1J, 4)
        s_j = jax.nn.sigmoid(p['l%d_job_skip' % l]).reshape(1, 1)
        h_job = _post(h_job, p['l%d_job_Wa' % l], p['l%d_job_ba' % l], s_j,
                      [num_s, den_s, num_a, den_a], ng=4)
        if l == 0:
            num_r, den_r = _edge_conv(q_u, k_r, v_r, si_rev, di_rev, N1J, N1U, 2)
            s_u = jax.nn.sigmoid(p['l%d_user_skip' % l]).reshape(1, 1)
            h_user = _post(h_user, p['l%d_user_Wa' % l], p['l%d_user_ba' % l],
                           s_u, [num_r, den_r], ng=2)

    w1a, w1b = p['pred_W1'][:OUT], p['pred_W1'][OUT:]
    wg1 = p['out_W'] @ w1a
    wg2 = p['out_W'] @ w1b
    btot = p['out_b'] @ w1a + p['out_b'] @ w1b + p['pred_b1']
    g1, g2 = _mm(h_job, jnp.concatenate([wg1, wg2], axis=1),
                 jnp.zeros((2 * HID,), F32), nout=2)

    epl = 204800
    nlab = edge_label_index.shape[1]
    src = jnp.concatenate([edge_label_index[0], jnp.full((epl - nlab,), NJOB, I32)])
    dst = jnp.concatenate([edge_label_index[1], jnp.full((epl - nlab,), NJOB, I32)])
    ga, gb = _make_pair_gather(epl, N1J)(g1, g2, src, dst)
    pred = _pred(ga, gb, btot, p['pred_W2'], p['pred_b2'])
    return pred[:nlab, 0]
